# Initial kernel scaffold; baseline (speedup 1.0000x reference)
import jax
import jax.numpy as jnp
from jax.experimental import pallas as pl


def kernel(X, pos_enc, adj, atom_emb, w32, w64, w128, bias):
    raise NotImplementedError("write your pallas kernel here")


# speedup vs baseline: 2.1130x; 2.1130x over previous
import jax
import jax.numpy as jnp
from jax.experimental import pallas as pl
from jax.experimental.pallas import tpu as pltpu

_N = 16
_LOG2N = 4
_H = 32
_NH = 4
_NL = 2
_F = 64
_G = 8
_M = _G * _N
_EPS = 1e-5
_NEG = -1e30
_PB = 128


def _gt_kernel(h0_ref, pos_ref, adj_ref, wqkv_ref, wpos_ref, wo_ref,
               w1_ref, w2_ref, b_ref, out_ref):
    f32 = jnp.float32

    ri = jax.lax.broadcasted_iota(jnp.int32, (_M, _M), 0) >> _LOG2N
    ci = jax.lax.broadcasted_iota(jnp.int32, (_M, _M), 1) >> _LOG2N
    sameg = ri == ci
    lane_head = jax.lax.broadcasted_iota(jnp.int32, (1, _H), 1) >> 3
    rg = jax.lax.broadcasted_iota(jnp.int32, (_G, _M), 0)
    cg = jax.lax.broadcasted_iota(jnp.int32, (_G, _M), 1) >> _LOG2N
    poolm = (rg == cg).astype(f32)
    meanm = poolm * (1.0 / _N)
    er = jax.lax.broadcasted_iota(jnp.int32, (_M, _G), 0) >> _LOG2N
    ec = jax.lax.broadcasted_iota(jnp.int32, (_M, _G), 1)
    expandm = (er == ec).astype(f32)

    a = adj_ref[0]
    a8 = jnp.concatenate([a] * _G, axis=1)
    amask = jnp.where(sameg & (a8 > 0.0), 0.0, _NEG)

    def brow(r, width):
        return b_ref[r:r + 1, :width]

    def bn(x, g_row, b_row):
        st = jnp.dot(meanm, jnp.concatenate([x, x * x], axis=1),
                     preferred_element_type=f32)
        mu = st[:, :_H]
        var = jnp.maximum(st[:, _H:] - mu * mu, 0.0)
        inv = jax.lax.rsqrt(var + _EPS)
        mi = jnp.dot(expandm, jnp.concatenate([mu, inv], axis=1),
                     preferred_element_type=f32)
        return (x - mi[:, :_H]) * mi[:, _H:] * g_row + b_row

    h = (h0_ref[...]
         + jnp.dot(pos_ref[0], wpos_ref[0:2, :], preferred_element_type=f32)
         + brow(0, _H))

    for l in range(_NL):
        bb = 1 + l * 8

        qkv = jnp.dot(h, wqkv_ref[l * _H:(l + 1) * _H, :],
                      preferred_element_type=f32)
        q = qkv[:, 0:_H] + brow(bb, _H)
        k = qkv[:, _H:2 * _H]
        v = qkv[:, 2 * _H:3 * _H]
        zer = jnp.zeros_like(k)
        kst = jnp.concatenate(
            [jnp.where(lane_head == hd, k, zer) for hd in range(_NH)],
            axis=0)
        s_all = jax.lax.dot_general(q, kst, (((1,), (1,)), ((), ())),
                                    preferred_element_type=f32)
        ps = []
        for hd in range(_NH):
            sh = s_all[:, hd * _M:(hd + 1) * _M] + amask
            m = jnp.max(sh, axis=-1, keepdims=True)
            p = jnp.exp(sh - m)
            ps.append(p * pl.reciprocal(jnp.sum(p, axis=-1, keepdims=True),
                                        approx=True))
        attn = jnp.concatenate(ps, axis=1)
        vst = jnp.concatenate(
            [jnp.where(lane_head == hd, v, zer) for hd in range(_NH)],
            axis=0)
        ctx = jnp.dot(attn, vst, preferred_element_type=f32)
        mha = (jnp.dot(ctx, wo_ref[l * _H:(l + 1) * _H, :],
                       preferred_element_type=f32) + brow(bb + 1, _H))

        x = bn(mha + h, brow(bb + 2, _H), brow(bb + 3, _H))

        f = jnp.maximum(jnp.dot(x, w1_ref[l * _H:(l + 1) * _H, :],
                                preferred_element_type=f32)
                        + brow(bb + 4, _F), 0.0)
        f2 = (jnp.dot(f, w2_ref[l * _F:(l + 1) * _F, :],
                      preferred_element_type=f32) + brow(bb + 5, _H))

        h = bn(x + f2, brow(bb + 6, _H), brow(bb + 7, _H))

    out_ref[...] = jnp.dot(poolm, h, preferred_element_type=f32)


def _pred_kernel(pool_ref, w128_ref, b_ref, out_ref):
    f32 = jnp.float32
    b1 = 1 + _NL * 11
    z = jnp.maximum(jnp.dot(pool_ref[...], w128_ref[0:_H, :],
                            preferred_element_type=f32)
                    + b_ref[b1:b1 + 1, :], 0.0)
    z = jnp.maximum(jnp.dot(z, w128_ref[_H:_H + 128, :],
                            preferred_element_type=f32)
                    + b_ref[b1 + 1:b1 + 2, :], 0.0)
    out_ref[...] = (jnp.dot(z, w128_ref[_H + 128:_H + 256, :],
                            preferred_element_type=f32)
                    + b_ref[b1 + 2:b1 + 3, :])


def kernel(X, pos_enc, adj, atom_emb, w32, w64, w128, bias):
    f32 = jnp.float32
    num_graphs = X.shape[0]
    num_steps = num_graphs // _G

    table = (atom_emb[0][:, None, None, :] + atom_emb[1][None, :, None, :]
             + atom_emb[2][None, None, :, :]).reshape(512, _H)
    idx = (X[..., 0] * 64 + X[..., 1] * 8 + X[..., 2]).reshape(-1)
    h0 = jnp.take(table, idx, axis=0)

    pos3 = pos_enc.astype(f32).reshape(num_steps, _M, 2)
    adj3 = adj.reshape(num_steps, _M, _N)

    wpos = w32[0:8, 0:_H]
    wqkv_l, wo_l, w2_l, brows = [], [], [], []
    brows.append(jnp.pad(bias[0, 0:_H], (0, _F - _H)))
    for l in range(_NL):
        base = 8 + l * (4 * _NH * _H + _F)
        bb = 1 + l * 11
        wq_h, wk_h, wv_h, wo_h = [], [], [], []
        bq_h = []
        for hd in range(_NH):
            hb = base + hd * 4 * _H
            wq_h.append(w32[hb:hb + _H, 0:8])
            wk_h.append(w32[hb + _H:hb + 2 * _H, 0:8])
            wv_h.append(w32[hb + 2 * _H:hb + 3 * _H, 0:8])
            wo_h.append(w32[hb + 3 * _H:hb + 3 * _H + 8, 0:_H])
            bq_h.append(bias[bb + hd, 0:8])
        wqkv_l.append(jnp.concatenate(
            wq_h + wk_h + wv_h + [jnp.zeros((_H, _H), f32)], axis=1))
        wo_l.append(jnp.concatenate(wo_h, axis=0))
        w2_l.append(w32[base + 4 * _NH * _H:base + 4 * _NH * _H + _F, 0:_H])
        brows.append(jnp.pad(jnp.concatenate(bq_h), (0, _H)))
        brows.append(jnp.pad(bias[bb + 4, 0:_H], (0, _H)))
        brows.append(jnp.pad(bias[bb + 5, 0:_H], (0, _H)))
        brows.append(jnp.pad(bias[bb + 6, 0:_H], (0, _H)))
        brows.append(bias[bb + 7, 0:_F])
        brows.append(jnp.pad(bias[bb + 8, 0:_H], (0, _H)))
        brows.append(jnp.pad(bias[bb + 9, 0:_H], (0, _H)))
        brows.append(jnp.pad(bias[bb + 10, 0:_H], (0, _H)))
    wqkv = jnp.concatenate(wqkv_l, axis=0)
    wo = jnp.concatenate(wo_l, axis=0)
    w2 = jnp.concatenate(w2_l, axis=0)
    bvec = jnp.stack(brows, axis=0)
    bvec = jnp.pad(bvec, ((0, 24 - bvec.shape[0]), (0, 0)))

    pooled = pl.pallas_call(
        _gt_kernel,
        grid=(num_steps,),
        in_specs=[
            pl.BlockSpec((_M, _H), lambda s: (s, 0)),
            pl.BlockSpec((1, _M, 2), lambda s: (s, 0, 0)),
            pl.BlockSpec((1, _M, _N), lambda s: (s, 0, 0)),
            pl.BlockSpec((_NL * _H, 128), lambda s: (0, 0)),
            pl.BlockSpec((8, _H), lambda s: (0, 0)),
            pl.BlockSpec((_NL * _H, _H), lambda s: (0, 0)),
            pl.BlockSpec((_NL * _H, _F), lambda s: (0, 0)),
            pl.BlockSpec((_NL * _F, _H), lambda s: (0, 0)),
            pl.BlockSpec((24, _F), lambda s: (0, 0)),
        ],
        out_specs=pl.BlockSpec((_G, _H), lambda s: (s, 0)),
        out_shape=jax.ShapeDtypeStruct((num_graphs, _H), f32),
        compiler_params=pltpu.CompilerParams(
            dimension_semantics=("parallel",)),
    )(h0, pos3, adj3, wqkv, wpos, wo, w64, w2, bvec)

    out_pad = pl.pallas_call(
        _pred_kernel,
        grid=(pl.cdiv(num_graphs, _PB),),
        in_specs=[
            pl.BlockSpec((_PB, _H), lambda s: (s, 0)),
            pl.BlockSpec((_H + 256, 128), lambda s: (0, 0)),
            pl.BlockSpec((32, 128), lambda s: (0, 0)),
        ],
        out_specs=pl.BlockSpec((_PB, 128), lambda s: (s, 0)),
        out_shape=jax.ShapeDtypeStruct((num_graphs, 128), f32),
        compiler_params=pltpu.CompilerParams(
            dimension_semantics=("parallel",)),
    )(pooled, w128, bias)

    return out_pad[:, :4]


# speedup vs baseline: 4.3767x; 2.0713x over previous
import jax
import jax.numpy as jnp
from jax.experimental import pallas as pl
from jax.experimental.pallas import tpu as pltpu

_N = 16
_LOG2N = 4
_H = 32
_NH = 4
_NL = 2
_F = 64
_G = 8
_M = _G * _N
_EPS = 1e-5
_NEG = -1e30
_PB = 128
_CH = 16


def _gt_kernel(h0_ref, pos_ref, adj_ref, wqkv_ref, wpos_ref, wo_ref,
               w1_ref, w2_ref, b_ref, out_ref):
    f32 = jnp.float32

    ri = jax.lax.broadcasted_iota(jnp.int32, (_M, _M), 0) >> _LOG2N
    ci = jax.lax.broadcasted_iota(jnp.int32, (_M, _M), 1) >> _LOG2N
    sameg = ri == ci
    lane_head = jax.lax.broadcasted_iota(jnp.int32, (1, _H), 1) >> 3
    rg = jax.lax.broadcasted_iota(jnp.int32, (_G, _M), 0)
    cg = jax.lax.broadcasted_iota(jnp.int32, (_G, _M), 1) >> _LOG2N
    poolm = (rg == cg).astype(f32)
    meanm = poolm * (1.0 / _N)
    er = jax.lax.broadcasted_iota(jnp.int32, (_M, _G), 0) >> _LOG2N
    ec = jax.lax.broadcasted_iota(jnp.int32, (_M, _G), 1)
    expandm = (er == ec).astype(f32)

    def brow(r, width):
        return b_ref[r:r + 1, :width]

    def bn_stats(x):
        st = jnp.dot(meanm, jnp.concatenate([x, x * x], axis=1),
                     preferred_element_type=f32)
        mu = st[:, :_H]
        var = jnp.maximum(st[:, _H:] - mu * mu, 0.0)
        inv = jax.lax.rsqrt(var + _EPS)
        return jnp.dot(expandm, jnp.concatenate([mu, inv], axis=1),
                       preferred_element_type=f32)

    def bn_apply(x, mi, g_row, b_row):
        return (x - mi[:, :_H]) * mi[:, _H:] * g_row + b_row

    CH = range(_CH)

    amask = []
    for c in CH:
        a = adj_ref[0, c * _M:(c + 1) * _M, :]
        a8 = jnp.concatenate([a] * _G, axis=1)
        amask.append(jnp.where(sameg & (a8 > 0.0), 0.0, _NEG))

    h = [(h0_ref[c * _M:(c + 1) * _M, :]
          + jnp.dot(pos_ref[0, c * _M:(c + 1) * _M, :], wpos_ref[0:2, :],
                    preferred_element_type=f32)
          + brow(0, _H)) for c in CH]

    for l in range(_NL):
        bb = 1 + l * 8
        wqkv = wqkv_ref[l * _H:(l + 1) * _H, :]
        wo = wo_ref[l * _H:(l + 1) * _H, :]
        w1 = w1_ref[l * _H:(l + 1) * _H, :]
        w2 = w2_ref[l * _F:(l + 1) * _F, :]

        qkv = [jnp.dot(h[c], wqkv, preferred_element_type=f32) for c in CH]
        q = [qkv[c][:, 0:_H] + brow(bb, _H) for c in CH]
        k = [qkv[c][:, _H:2 * _H] for c in CH]
        v = [qkv[c][:, 2 * _H:3 * _H] for c in CH]
        zer = jnp.zeros((_M, _H), f32)
        kst = [jnp.concatenate(
            [jnp.where(lane_head == hd, k[c], zer) for hd in range(_NH)],
            axis=0) for c in CH]
        s_all = [jax.lax.dot_general(q[c], kst[c], (((1,), (1,)), ((), ())),
                                     preferred_element_type=f32)
                 for c in CH]
        sh = [[s_all[c][:, hd * _M:(hd + 1) * _M] + amask[c]
               for hd in range(_NH)] for c in CH]
        mx = [[jnp.max(sh[c][hd], axis=-1, keepdims=True)
               for hd in range(_NH)] for c in CH]
        p = [[jnp.exp(sh[c][hd] - mx[c][hd]) for hd in range(_NH)]
             for c in CH]
        attn = [jnp.concatenate(
            [p[c][hd] * pl.reciprocal(jnp.sum(p[c][hd], axis=-1,
                                              keepdims=True), approx=True)
             for hd in range(_NH)], axis=1) for c in CH]
        vst = [jnp.concatenate(
            [jnp.where(lane_head == hd, v[c], zer) for hd in range(_NH)],
            axis=0) for c in CH]
        ctx = [jnp.dot(attn[c], vst[c], preferred_element_type=f32)
               for c in CH]
        mha = [jnp.dot(ctx[c], wo, preferred_element_type=f32)
               + brow(bb + 1, _H) for c in CH]

        xr = [mha[c] + h[c] for c in CH]
        mi = [bn_stats(xr[c]) for c in CH]
        x = [bn_apply(xr[c], mi[c], brow(bb + 2, _H), brow(bb + 3, _H))
             for c in CH]

        f = [jnp.maximum(jnp.dot(x[c], w1, preferred_element_type=f32)
                         + brow(bb + 4, _F), 0.0) for c in CH]
        f2 = [jnp.dot(f[c], w2, preferred_element_type=f32)
              + brow(bb + 5, _H) for c in CH]

        yr = [x[c] + f2[c] for c in CH]
        mi2 = [bn_stats(yr[c]) for c in CH]
        h = [bn_apply(yr[c], mi2[c], brow(bb + 6, _H), brow(bb + 7, _H))
             for c in CH]

    for c in CH:
        out_ref[c * _G:(c + 1) * _G, :] = jnp.dot(
            poolm, h[c], preferred_element_type=f32)


def _pred_kernel(pool_ref, w128_ref, b_ref, out_ref):
    f32 = jnp.float32
    b1 = 1 + _NL * 11
    z = jnp.maximum(jnp.dot(pool_ref[...], w128_ref[0:_H, :],
                            preferred_element_type=f32)
                    + b_ref[b1:b1 + 1, :], 0.0)
    z = jnp.maximum(jnp.dot(z, w128_ref[_H:_H + 128, :],
                            preferred_element_type=f32)
                    + b_ref[b1 + 1:b1 + 2, :], 0.0)
    out_ref[...] = (jnp.dot(z, w128_ref[_H + 128:_H + 256, :],
                            preferred_element_type=f32)
                    + b_ref[b1 + 2:b1 + 3, :])


def kernel(X, pos_enc, adj, atom_emb, w32, w64, w128, bias):
    f32 = jnp.float32
    num_graphs = X.shape[0]

    table = (atom_emb[0][:, None, None, :] + atom_emb[1][None, :, None, :]
             + atom_emb[2][None, None, :, :]).reshape(512, _H)
    idx = (X[..., 0] * 64 + X[..., 1] * 8 + X[..., 2]).reshape(-1)
    h0 = jnp.take(table, idx, axis=0)

    num_steps = num_graphs // (_G * _CH)
    pos3 = pos_enc.astype(f32).reshape(num_steps, _CH * _M, 2)
    adj3 = adj.reshape(num_steps, _CH * _M, _N)

    wpos = w32[0:8, 0:_H]
    wqkv_l, wo_l, w2_l, brows = [], [], [], []
    brows.append(jnp.pad(bias[0, 0:_H], (0, _F - _H)))
    for l in range(_NL):
        base = 8 + l * (4 * _NH * _H + _F)
        bb = 1 + l * 11
        wq_h, wk_h, wv_h, wo_h = [], [], [], []
        bq_h = []
        for hd in range(_NH):
            hb = base + hd * 4 * _H
            wq_h.append(w32[hb:hb + _H, 0:8])
            wk_h.append(w32[hb + _H:hb + 2 * _H, 0:8])
            wv_h.append(w32[hb + 2 * _H:hb + 3 * _H, 0:8])
            wo_h.append(w32[hb + 3 * _H:hb + 3 * _H + 8, 0:_H])
            bq_h.append(bias[bb + hd, 0:8])
        wqkv_l.append(jnp.concatenate(
            wq_h + wk_h + wv_h + [jnp.zeros((_H, _H), f32)], axis=1))
        wo_l.append(jnp.concatenate(wo_h, axis=0))
        w2_l.append(w32[base + 4 * _NH * _H:base + 4 * _NH * _H + _F, 0:_H])
        brows.append(jnp.pad(jnp.concatenate(bq_h), (0, _H)))
        brows.append(jnp.pad(bias[bb + 4, 0:_H], (0, _H)))
        brows.append(jnp.pad(bias[bb + 5, 0:_H], (0, _H)))
        brows.append(jnp.pad(bias[bb + 6, 0:_H], (0, _H)))
        brows.append(bias[bb + 7, 0:_F])
        brows.append(jnp.pad(bias[bb + 8, 0:_H], (0, _H)))
        brows.append(jnp.pad(bias[bb + 9, 0:_H], (0, _H)))
        brows.append(jnp.pad(bias[bb + 10, 0:_H], (0, _H)))
    wqkv = jnp.concatenate(wqkv_l, axis=0)
    wo = jnp.concatenate(wo_l, axis=0)
    w2 = jnp.concatenate(w2_l, axis=0)
    bvec = jnp.stack(brows, axis=0)
    bvec = jnp.pad(bvec, ((0, 24 - bvec.shape[0]), (0, 0)))

    pooled = pl.pallas_call(
        _gt_kernel,
        grid=(num_steps,),
        in_specs=[
            pl.BlockSpec((_CH * _M, _H), lambda s: (s, 0)),
            pl.BlockSpec((1, _CH * _M, 2), lambda s: (s, 0, 0)),
            pl.BlockSpec((1, _CH * _M, _N), lambda s: (s, 0, 0)),
            pl.BlockSpec((_NL * _H, 128), lambda s: (0, 0)),
            pl.BlockSpec((8, _H), lambda s: (0, 0)),
            pl.BlockSpec((_NL * _H, _H), lambda s: (0, 0)),
            pl.BlockSpec((_NL * _H, _F), lambda s: (0, 0)),
            pl.BlockSpec((_NL * _F, _H), lambda s: (0, 0)),
            pl.BlockSpec((24, _F), lambda s: (0, 0)),
        ],
        out_specs=pl.BlockSpec((_CH * _G, _H), lambda s: (s, 0)),
        out_shape=jax.ShapeDtypeStruct((num_graphs, _H), f32),
        compiler_params=pltpu.CompilerParams(
            dimension_semantics=("parallel",)),
    )(h0, pos3, adj3, wqkv, wpos, wo, w64, w2, bvec)

    out_pad = pl.pallas_call(
        _pred_kernel,
        grid=(pl.cdiv(num_graphs, _PB),),
        in_specs=[
            pl.BlockSpec((_PB, _H), lambda s: (s, 0)),
            pl.BlockSpec((_H + 256, 128), lambda s: (0, 0)),
            pl.BlockSpec((32, 128), lambda s: (0, 0)),
        ],
        out_specs=pl.BlockSpec((_PB, 128), lambda s: (s, 0)),
        out_shape=jax.ShapeDtypeStruct((num_graphs, 128), f32),
        compiler_params=pltpu.CompilerParams(
            dimension_semantics=("parallel",)),
    )(pooled, w128, bias)

    return out_pad[:, :4]


# speedup vs baseline: 6.7240x; 1.5363x over previous
import jax
import jax.numpy as jnp
from jax.experimental import pallas as pl
from jax.experimental.pallas import tpu as pltpu

_N = 16
_LOG2N = 4
_H = 32
_NH = 4
_NL = 2
_F = 64
_G = 8
_M = _G * _N
_EPS = 1e-5
_NEG = -1e30
_PB = 128
_CH = 16


def _gt_kernel(h0_ref, pos_ref, adj_ref, wqkv_ref, wpos_ref,
               wo_ref, w1_ref, w2_ref, b_ref, out_ref):
    f32 = jnp.float32

    ri = jax.lax.broadcasted_iota(jnp.int32, (_M, _M), 0) >> _LOG2N
    ci = jax.lax.broadcasted_iota(jnp.int32, (_M, _M), 1) >> _LOG2N
    sameg = ri == ci
    lane_head = jax.lax.broadcasted_iota(jnp.int32, (1, _H), 1) >> 3
    rg = jax.lax.broadcasted_iota(jnp.int32, (_G, _M), 0)
    cg = jax.lax.broadcasted_iota(jnp.int32, (_G, _M), 1) >> _LOG2N
    poolm = (rg == cg).astype(f32)
    meanm = poolm * (1.0 / _N)
    er = jax.lax.broadcasted_iota(jnp.int32, (_M, _G), 0) >> _LOG2N
    ec = jax.lax.broadcasted_iota(jnp.int32, (_M, _G), 1)
    expandm = (er == ec).astype(f32)

    def brow(r, width):
        return b_ref[r:r + 1, :width]

    CH = range(_CH)
    pr = jax.lax.broadcasted_iota(jnp.int32, (_CH * _G, _CH * _M), 0)
    pc = jax.lax.broadcasted_iota(jnp.int32, (_CH * _G, _CH * _M), 1) >> _LOG2N
    poolbig = (pr == pc).astype(f32)
    meanbig = poolbig * (1.0 / _N)

    def bn_stats(xs):
        ycat = jnp.concatenate(
            [jnp.concatenate([x, x * x], axis=1) for x in xs], axis=0)
        stb = jnp.dot(meanbig, ycat, preferred_element_type=f32)
        mu_all = stb[:, :_H]
        var_all = jnp.maximum(stb[:, _H:] - mu_all * mu_all, 0.0)
        inv_all = jax.lax.rsqrt(var_all + _EPS)
        return [jnp.dot(expandm, jnp.concatenate(
            [mu_all[c * _G:(c + 1) * _G, :],
             inv_all[c * _G:(c + 1) * _G, :]], axis=1),
            preferred_element_type=f32) for c in CH]

    def bn_apply(x, mi, g_row, b_row):
        return (x - mi[:, :_H]) * mi[:, _H:] * g_row + b_row

    h = [(h0_ref[c * _M:(c + 1) * _M, :]
          + jnp.dot(pos_ref[0, c * _M:(c + 1) * _M, :], wpos_ref[0:2, :],
                    preferred_element_type=f32)
          + brow(0, _H)) for c in CH]

    amask = []
    for c in CH:
        a = adj_ref[0, c * _M:(c + 1) * _M, :]
        a8 = jnp.concatenate([a] * _G, axis=1)
        amask.append(jnp.where(sameg & (a8 > 0.0), 0.0, _NEG))

    for l in range(_NL):
        bb = 1 + l * 8
        wqkv = wqkv_ref[l * _H:(l + 1) * _H, :]
        wo = wo_ref[l * _H:(l + 1) * _H, :]
        w1 = w1_ref[l * _H:(l + 1) * _H, :]
        w2 = w2_ref[l * _F:(l + 1) * _F, :]

        qkv = [jnp.dot(h[c], wqkv, preferred_element_type=f32) for c in CH]
        q = [qkv[c][:, 0:_H] + brow(bb, _H) for c in CH]
        k = [qkv[c][:, _H:2 * _H] for c in CH]
        v = [qkv[c][:, 2 * _H:3 * _H] for c in CH]
        zer = jnp.zeros((_M, _H), f32)
        kst = [[jnp.concatenate(
            [jnp.where(lane_head == 2 * j + i, k[c], zer) for i in range(2)],
            axis=0) for j in range(2)] for c in CH]
        s_all = [[jax.lax.dot_general(q[c], kst[c][j],
                                      (((1,), (1,)), ((), ())),
                                      preferred_element_type=f32)
                  for j in range(2)] for c in CH]
        sh = [[s_all[c][hd // 2][:, (hd % 2) * _M:(hd % 2 + 1) * _M]
               + amask[c] for hd in range(_NH)] for c in CH]
        mx = [[jnp.max(sh[c][hd], axis=-1, keepdims=True)
               for hd in range(_NH)] for c in CH]
        p = [[jnp.exp(sh[c][hd] - mx[c][hd]) for hd in range(_NH)]
             for c in CH]
        pn = [[p[c][hd] * pl.reciprocal(jnp.sum(p[c][hd], axis=-1,
                                                keepdims=True), approx=True)
               for hd in range(_NH)] for c in CH]
        vst = [[jnp.concatenate(
            [jnp.where(lane_head == 2 * j + i, v[c], zer) for i in range(2)],
            axis=0) for j in range(2)] for c in CH]
        ctx = [jnp.dot(jnp.concatenate(pn[c][0:2], axis=1), vst[c][0],
                       preferred_element_type=f32)
               + jnp.dot(jnp.concatenate(pn[c][2:4], axis=1), vst[c][1],
                         preferred_element_type=f32) for c in CH]
        mha = [jnp.dot(ctx[c], wo, preferred_element_type=f32)
               + brow(bb + 1, _H) for c in CH]

        xr = [mha[c] + h[c] for c in CH]
        mi = bn_stats(xr)
        x = [bn_apply(xr[c], mi[c], brow(bb + 2, _H), brow(bb + 3, _H))
             for c in CH]

        f = [jnp.maximum(jnp.dot(x[c], w1, preferred_element_type=f32)
                         + brow(bb + 4, _F), 0.0) for c in CH]
        f2 = [jnp.dot(f[c], w2, preferred_element_type=f32)
              + brow(bb + 5, _H) for c in CH]

        yr = [x[c] + f2[c] for c in CH]
        mi2 = bn_stats(yr)
        h = [bn_apply(yr[c], mi2[c], brow(bb + 6, _H), brow(bb + 7, _H))
             for c in CH]

    out_ref[...] = jnp.dot(poolbig, jnp.concatenate(h, axis=0),
                           preferred_element_type=f32)


def _pred_kernel(pool_ref, w128_ref, b_ref, out_ref):
    f32 = jnp.float32
    b1 = 1 + _NL * 11
    z = jnp.maximum(jnp.dot(pool_ref[...], w128_ref[0:_H, :],
                            preferred_element_type=f32)
                    + b_ref[b1:b1 + 1, :], 0.0)
    z = jnp.maximum(jnp.dot(z, w128_ref[_H:_H + 128, :],
                            preferred_element_type=f32)
                    + b_ref[b1 + 1:b1 + 2, :], 0.0)
    out_ref[...] = (jnp.dot(z, w128_ref[_H + 128:_H + 256, :],
                            preferred_element_type=f32)
                    + b_ref[b1 + 2:b1 + 3, :])


def kernel(X, pos_enc, adj, atom_emb, w32, w64, w128, bias):
    f32 = jnp.float32
    num_graphs = X.shape[0]

    table = (atom_emb[0][:, None, None, :] + atom_emb[1][None, :, None, :]
             + atom_emb[2][None, None, :, :]).reshape(512, _H)
    idx = (X[..., 0] * 64 + X[..., 1] * 8 + X[..., 2]).reshape(-1)
    h0 = jnp.take(table, idx, axis=0)

    num_steps = num_graphs // (_G * _CH)
    pos3 = pos_enc.astype(f32).reshape(num_steps, _CH * _M, 2)
    adj3 = adj.reshape(num_steps, _CH * _M, _N)

    wpos = w32[0:8, 0:_H]
    wqkv_l, wo_l, w2_l, brows = [], [], [], []
    brows.append(jnp.pad(bias[0, 0:_H], (0, _F - _H)))
    for l in range(_NL):
        base = 8 + l * (4 * _NH * _H + _F)
        bb = 1 + l * 11
        wq_h, wk_h, wv_h, wo_h = [], [], [], []
        bq_h = []
        for hd in range(_NH):
            hb = base + hd * 4 * _H
            wq_h.append(w32[hb:hb + _H, 0:8])
            wk_h.append(w32[hb + _H:hb + 2 * _H, 0:8])
            wv_h.append(w32[hb + 2 * _H:hb + 3 * _H, 0:8])
            wo_h.append(w32[hb + 3 * _H:hb + 3 * _H + 8, 0:_H])
            bq_h.append(bias[bb + hd, 0:8])
        wqkv_l.append(jnp.concatenate(
            wq_h + wk_h + wv_h + [jnp.zeros((_H, _H), f32)], axis=1))
        wo_l.append(jnp.concatenate(wo_h, axis=0))
        w2_l.append(w32[base + 4 * _NH * _H:base + 4 * _NH * _H + _F, 0:_H])
        brows.append(jnp.pad(jnp.concatenate(bq_h), (0, _H)))
        brows.append(jnp.pad(bias[bb + 4, 0:_H], (0, _H)))
        brows.append(jnp.pad(bias[bb + 5, 0:_H], (0, _H)))
        brows.append(jnp.pad(bias[bb + 6, 0:_H], (0, _H)))
        brows.append(bias[bb + 7, 0:_F])
        brows.append(jnp.pad(bias[bb + 8, 0:_H], (0, _H)))
        brows.append(jnp.pad(bias[bb + 9, 0:_H], (0, _H)))
        brows.append(jnp.pad(bias[bb + 10, 0:_H], (0, _H)))
    wqkv = jnp.concatenate(wqkv_l, axis=0)
    wo = jnp.concatenate(wo_l, axis=0)
    w2 = jnp.concatenate(w2_l, axis=0)
    bvec = jnp.stack(brows, axis=0)
    bvec = jnp.pad(bvec, ((0, 24 - bvec.shape[0]), (0, 0)))

    pooled = pl.pallas_call(
        _gt_kernel,
        grid=(num_steps,),
        in_specs=[
            pl.BlockSpec((_CH * _M, _H), lambda s: (s, 0)),
            pl.BlockSpec((1, _CH * _M, 2), lambda s: (s, 0, 0)),
            pl.BlockSpec((1, _CH * _M, _N), lambda s: (s, 0, 0)),
            pl.BlockSpec((_NL * _H, 128), lambda s: (0, 0)),
            pl.BlockSpec((8, _H), lambda s: (0, 0)),
            pl.BlockSpec((_NL * _H, _H), lambda s: (0, 0)),
            pl.BlockSpec((_NL * _H, _F), lambda s: (0, 0)),
            pl.BlockSpec((_NL * _F, _H), lambda s: (0, 0)),
            pl.BlockSpec((24, _F), lambda s: (0, 0)),
        ],
        out_specs=pl.BlockSpec((_CH * _G, _H), lambda s: (s, 0)),
        out_shape=jax.ShapeDtypeStruct((num_graphs, _H), f32),
        compiler_params=pltpu.CompilerParams(
            dimension_semantics=("parallel",)),
    )(h0, pos3, adj3, wqkv, wpos, wo, w64, w2, bvec)

    out_pad = pl.pallas_call(
        _pred_kernel,
        grid=(pl.cdiv(num_graphs, _PB),),
        in_specs=[
            pl.BlockSpec((_PB, _H), lambda s: (s, 0)),
            pl.BlockSpec((_H + 256, 128), lambda s: (0, 0)),
            pl.BlockSpec((32, 128), lambda s: (0, 0)),
        ],
        out_specs=pl.BlockSpec((_PB, 128), lambda s: (s, 0)),
        out_shape=jax.ShapeDtypeStruct((num_graphs, 128), f32),
        compiler_params=pltpu.CompilerParams(
            dimension_semantics=("parallel",)),
    )(pooled, w128, bias)

    return out_pad[:, :4]
